# baseline (speedup 1.0000x reference)
import jax
import jax.numpy as jnp
from jax import lax
from jax.experimental import pallas as pl
from jax.experimental.pallas import tpu as pltpu
from jax.experimental.pallas import tpu_sc as plsc

_B, _S, _D = 4, 2048, 768
_NC, _NS = 2, 16
_NW = _NC * _NS
_ROWS_PER_W = _S // _NW
_K = 32
_VECS = _K * _D // 16


def _sc_body(x_hbm, pos_hbm, out_hbm, p_buf, x_buf):
    w = lax.axis_index("s") * _NC + lax.axis_index("c")
    base = w * _ROWS_PER_W

    def win_loop(c, _):
        pbase = base + c * _K
        pltpu.sync_copy(pos_hbm.at[pl.ds(pbase, _K)], p_buf)

        def batch_loop(b, _):
            pltpu.sync_copy(x_hbm.at[b, pl.ds(pbase, _K)], x_buf)

            def add_loop(i, _):
                r = i // (_D // 16)
                col = (i % (_D // 16)) * 16
                x_buf[r, pl.ds(col, 16)] = (
                    x_buf[r, pl.ds(col, 16)] + p_buf[r, pl.ds(col, 16)]
                )
                return 0

            lax.fori_loop(0, _VECS, add_loop, 0)
            pltpu.sync_copy(x_buf, out_hbm.at[b, pl.ds(pbase, _K)])
            return 0

        lax.fori_loop(0, _B, batch_loop, 0)
        return 0

    lax.fori_loop(0, _ROWS_PER_W // _K, win_loop, 0)


def kernel(inputs, pos_table):
    return pl.kernel(
        _sc_body,
        out_type=jax.ShapeDtypeStruct((_B, _S, _D), jnp.float32),
        mesh=plsc.VectorSubcoreMesh(core_axis_name="c", subcore_axis_name="s"),
        scratch_types=[
            pltpu.VMEM((_K, _D), jnp.float32),
            pltpu.VMEM((_K, _D), jnp.float32),
        ],
    )(inputs, pos_table)


# speedup vs baseline: 1.9825x; 1.9825x over previous
import jax
import jax.numpy as jnp
from jax import lax
from jax.experimental import pallas as pl
from jax.experimental.pallas import tpu as pltpu
from jax.experimental.pallas import tpu_sc as plsc

_B, _S, _D = 4, 2048, 768
_NC, _NS = 2, 16
_NW = _NC * _NS
_ROWS_PER_W = _S // _NW
_K = 32
_NV = _D // 16


def _sc_body(x_hbm, pos_hbm, out_hbm, p_buf0, p_buf1, x_buf0, x_buf1,
             sem_x0, sem_x1):
    w = lax.axis_index("s") * _NC + lax.axis_index("c")
    base = w * _ROWS_PER_W

    p_bufs = (p_buf0, p_buf1)
    x_bufs = (x_buf0, x_buf1)
    sems_x = (sem_x0, sem_x1)

    pltpu.sync_copy(pos_hbm.at[pl.ds(base, _K)], p_buf0)
    pltpu.sync_copy(pos_hbm.at[pl.ds(base + _K, _K)], p_buf1)

    seq = [(c, b) for c in range(2) for b in range(_B)]

    def start_load(g):
        c, b = seq[g]
        return pltpu.async_copy(
            x_hbm.at[b, pl.ds(base + c * _K, _K)], x_bufs[g % 2], sems_x[g % 2]
        )

    pending = start_load(0)
    for g, (c, b) in enumerate(seq):
        cur_copy = pending
        if g + 1 < len(seq):
            pending = start_load(g + 1)
        cur_copy.wait()

        p_buf = p_bufs[c]
        x_buf = x_bufs[g % 2]

        def add_row(r, _, x_buf=x_buf, p_buf=p_buf):
            for j in range(_NV):
                x_buf[r, pl.ds(j * 16, 16)] = (
                    x_buf[r, pl.ds(j * 16, 16)] + p_buf[r, pl.ds(j * 16, 16)]
                )
            return 0

        lax.fori_loop(0, _K, add_row, 0)
        pltpu.sync_copy(x_buf, out_hbm.at[b, pl.ds(base + c * _K, _K)])


def kernel(inputs, pos_table):
    return pl.kernel(
        _sc_body,
        out_type=jax.ShapeDtypeStruct((_B, _S, _D), jnp.float32),
        mesh=plsc.VectorSubcoreMesh(core_axis_name="c", subcore_axis_name="s"),
        scratch_types=[
            pltpu.VMEM((_K, _D), jnp.float32),
            pltpu.VMEM((_K, _D), jnp.float32),
            pltpu.VMEM((_K, _D), jnp.float32),
            pltpu.VMEM((_K, _D), jnp.float32),
            pltpu.SemaphoreType.DMA,
            pltpu.SemaphoreType.DMA,
        ],
    )(inputs, pos_table)
